# baseline (speedup 1.0000x reference)
import functools

import jax
import jax.numpy as jnp
from jax import lax
from jax.experimental import pallas as pl
from jax.experimental.pallas import tpu as pltpu
from jax.experimental.pallas import tpu_sc as plsc

N = 10000
E = 320000
F = 128
H = 128
L = 64
D = 256
B = 16
NC = 6

NPAD = 10112
PAD_ROW = 10008
NCORES = 2
NSUB = 16
NTILES = NCORES * NSUB
CHUNK = 128
NCHUNK = 80
HCHUNK = NCHUNK // 2
HPAIR = HCHUNK // 2
EPT = NCHUNK * CHUNK
EPAD = NTILES * EPT
RPT = NPAD // NSUB

_MESH = plsc.VectorSubcoreMesh(core_axis_name="c", subcore_axis_name="s")


def _deg_kernel(hi_hbm, oh_hbm, zeros_hbm, out_hbm, hi_v, oh_v, acc):
    cid = lax.axis_index("c")
    sid = lax.axis_index("s")
    wid = sid * NCORES + cid
    pltpu.sync_copy(hi_hbm.at[wid], hi_v)

    @pl.when(sid == 0)
    def _():
        pltpu.sync_copy(zeros_hbm, acc)

    plsc.subcore_barrier()

    def body(j, carry):
        pltpu.sync_copy(oh_hbm.at[wid, j], oh_v)
        pltpu.sync_copy(oh_v, acc.at[hi_v.at[j]], add=True)
        return carry

    lax.fori_loop(0, NCHUNK, body, 0)
    plsc.subcore_barrier()

    @pl.when(sid == 0)
    def _():
        pltpu.sync_copy(acc, out_hbm.at[cid])


_deg_call = pl.kernel(
    _deg_kernel,
    out_type=jax.ShapeDtypeStruct((NCORES, NPAD // 16, 16), jnp.float32),
    mesh=_MESH,
    scratch_types=[
        pltpu.VMEM((NCHUNK, CHUNK), jnp.int32),
        pltpu.VMEM((CHUNK, 16), jnp.float32),
        pltpu.VMEM_SHARED((NPAD // 16, 16), jnp.float32),
    ],
)


def _make_agg(width):
    def _agg_kernel(hp_hbm, src_hbm, dst_hbm, zeros_hbm, out_hbm,
                    src_v, dst_v, rows_a, rows_b, acc, sema):
        cid = lax.axis_index("c")
        sid = lax.axis_index("s")
        wid = sid * NCORES + cid
        r0 = sid * RPT
        pltpu.sync_copy(zeros_hbm.at[pl.ds(r0, RPT)], acc.at[pl.ds(r0, RPT)])
        plsc.subcore_barrier()

        for half in range(2):
            pltpu.sync_copy(src_hbm.at[wid, pl.ds(half * HCHUNK, HCHUNK)], src_v)
            pltpu.sync_copy(dst_hbm.at[wid, pl.ds(half * HCHUNK, HCHUNK)], dst_v)
            def body(j, carry):
                pltpu.async_copy(hp_hbm.at[src_v.at[j]], rows_a, sema).wait()
                pltpu.sync_copy(rows_a, acc.at[dst_v.at[j]], add=True)
                return carry

            lax.fori_loop(0, HCHUNK, body, 0)
        plsc.subcore_barrier()
        pltpu.sync_copy(acc.at[pl.ds(r0, RPT)], out_hbm.at[cid, pl.ds(r0, RPT)])

    return pl.kernel(
        _agg_kernel,
        out_type=jax.ShapeDtypeStruct((NCORES, NPAD, width), jnp.float32),
        mesh=_MESH,
        scratch_types=[
            pltpu.VMEM((HCHUNK, CHUNK), jnp.int32),
            pltpu.VMEM((HCHUNK, CHUNK), jnp.int32),
            pltpu.VMEM((CHUNK, width), jnp.float32),
            pltpu.VMEM((CHUNK, width), jnp.float32),
            pltpu.VMEM_SHARED((NPAD, width), jnp.float32),
            pltpu.SemaphoreType.DMA,
        ],
    )


_agg_h = _make_agg(H)


def _tc0_body(degp_ref, x_ref, w1_ref, dinv_ref, h1p_ref):
    ones2 = jnp.ones((NCORES, 1), jnp.float32)
    deg = lax.dot_general(degp_ref[...], ones2, (((0,), (0,)), ((), ())),
                          preferred_element_type=jnp.float32) + 1.0
    dinv = lax.rsqrt(deg)
    dinv_ref[...] = jnp.broadcast_to(dinv, (NPAD, 16))
    h1p_ref[...] = dinv * jnp.dot(x_ref[...], w1_ref[...],
                                  preferred_element_type=jnp.float32)


def _tc0(degp, x_p, w1):
    return pl.pallas_call(
        _tc0_body,
        out_shape=(
            jax.ShapeDtypeStruct((NPAD, 16), jnp.float32),
            jax.ShapeDtypeStruct((NPAD, F), jnp.float32),
        ),
    )(degp, x_p, w1)


def _tc_mid_body(sp_ref, hp_ref, dinv_ref, b_ref, g_ref, be_ref, w_ref, out_ref):
    dinv = dinv_ref[...][:, 0:1]
    pre = dinv * (sp_ref[0] + sp_ref[1] + hp_ref[...]) + b_ref[...]
    m = jnp.mean(pre[:N], axis=0, keepdims=True)
    c = pre - m
    v = jnp.mean(c[:N] * c[:N], axis=0, keepdims=True)
    a = jnp.maximum(c * lax.rsqrt(v + 1e-5) * g_ref[...] + be_ref[...], 0.0)
    mask = (lax.broadcasted_iota(jnp.int32, (NPAD, 1), 0) < N).astype(jnp.float32)
    out_ref[...] = dinv * jnp.dot(a * mask, w_ref[...],
                                  preferred_element_type=jnp.float32)


def _tc_mid(sp, hp, dinv16, b, g, be, w, width_out):
    return pl.pallas_call(
        _tc_mid_body,
        out_shape=jax.ShapeDtypeStruct((NPAD, width_out), jnp.float32),
    )(sp, hp, dinv16, b, g, be, w)


def _tc_fin_body(sp_ref, hp_ref, dinv_ref, b3_ref, g3_ref, be3_ref,
                 batch_ref, cam_ref, wl0a_ref, wl0b_ref, bl0_ref,
                 wl1_ref, bl1_ref, wl2_ref, bl2_ref, wout_ref, bout_ref,
                 out_ref):
    dinv = dinv_ref[...][:, 0:1]
    pre = dinv * (sp_ref[0] + sp_ref[1] + hp_ref[...]) + b3_ref[...]
    m = jnp.mean(pre[:N], axis=0, keepdims=True)
    c = pre - m
    v = jnp.mean(c[:N] * c[:N], axis=0, keepdims=True)
    a = c * lax.rsqrt(v + 1e-5) * g3_ref[...] + be3_ref[...]
    rid = lax.broadcasted_iota(jnp.int32, (B, NPAD), 0)
    p = (rid == batch_ref[...]).astype(jnp.float32)
    sums = jnp.dot(p, a, preferred_element_type=jnp.float32)
    cnt = jnp.sum(p, axis=1, keepdims=True)
    pooled = sums / jnp.maximum(cnt, 1.0)
    xd = jnp.maximum(
        jnp.dot(pooled, wl0a_ref[...], preferred_element_type=jnp.float32)
        + jnp.dot(cam_ref[...], wl0b_ref[...], preferred_element_type=jnp.float32)
        + bl0_ref[...], 0.0)
    xd = jnp.maximum(
        jnp.dot(xd, wl1_ref[...], preferred_element_type=jnp.float32)
        + bl1_ref[...], 0.0)
    xd = jnp.maximum(
        jnp.dot(xd, wl2_ref[...], preferred_element_type=jnp.float32)
        + bl2_ref[...], 0.0)
    out_ref[...] = (jnp.dot(xd, wout_ref[...], preferred_element_type=jnp.float32)
                    + bout_ref[...])


def _tc_fin(sp, hp, dinv16, b3, g3, be3, batch_p, cam,
            wl0a, wl0b, bl0, wl1, bl1, wl2, bl2, wout, bout):
    return pl.pallas_call(
        _tc_fin_body,
        out_shape=jax.ShapeDtypeStruct((B, 3), jnp.float32),
    )(sp, hp, dinv16, b3, g3, be3, batch_p, cam,
      wl0a, wl0b, bl0, wl1, bl1, wl2, bl2, wout, bout)


def kernel(x, edge_index, origin, direction, batch,
           W1, b1, g1, be1, W2, b2, g2, be2, W3, b3, g3, be3,
           Wl0, bl0, Wl1, bl1, Wl2, bl2, Wout, bout):
    src = edge_index[0].astype(jnp.int32)
    dst = edge_index[1].astype(jnp.int32)
    pad = jnp.full((EPAD - E,), PAD_ROW, jnp.int32)
    src3 = jnp.concatenate([src, pad]).reshape(NTILES, NCHUNK, CHUNK)
    dst3 = jnp.concatenate([dst, pad]).reshape(NTILES, NCHUNK, CHUNK)
    x_p = jnp.pad(x, ((0, NPAD - N), (0, 0)))
    batch_p = jnp.pad(batch.astype(jnp.int32), (0, NPAD - N),
                      constant_values=-1).reshape(1, NPAD)
    cam = jnp.concatenate([origin, direction], axis=1)
    zh = jnp.zeros((NPAD, H), jnp.float32)
    W3p = jnp.pad(W3, ((0, 0), (0, H - L)))
    b3p = jnp.pad(b3, (0, H - L)).reshape(1, H)
    g3p = jnp.pad(g3, (0, H - L)).reshape(1, H)
    be3p = jnp.pad(be3, (0, H - L)).reshape(1, H)
    wl0a = jnp.pad(Wl0[:L], ((0, H - L), (0, 0)))

    dst_p = jnp.concatenate([dst, pad])
    hi3 = lax.shift_right_logical(dst_p, 4).reshape(NTILES, NCHUNK, CHUNK)
    oh4 = jax.nn.one_hot(jnp.bitwise_and(dst_p, 15), 16, dtype=jnp.float32
                         ).reshape(NTILES, NCHUNK, CHUNK, 16)
    zdeg = jnp.zeros((NPAD // 16, 16), jnp.float32)
    degp = _deg_call(hi3, oh4, zdeg).reshape(NCORES, NPAD)
    dinv16, h1p = _tc0(degp, x_p, W1)
    s1 = _agg_h(h1p, src3, dst3, zh)
    h2p = _tc_mid(s1, h1p, dinv16, b1.reshape(1, H), g1.reshape(1, H),
                  be1.reshape(1, H), W2, H)
    s2 = _agg_h(h2p, src3, dst3, zh)
    h3p = _tc_mid(s2, h2p, dinv16, b2.reshape(1, H), g2.reshape(1, H),
                  be2.reshape(1, H), W3p, H)
    s3 = _agg_h(h3p, src3, dst3, zh)
    return _tc_fin(s3, h3p, dinv16, b3p, g3p, be3p, batch_p, cam,
                   wl0a, Wl0[L:], bl0.reshape(1, D),
                   Wl1, bl1.reshape(1, D), Wl2, bl2.reshape(1, D),
                   Wout, bout.reshape(1, 3))


# speedup vs baseline: 1.0400x; 1.0400x over previous
import functools

import jax
import jax.numpy as jnp
from jax import lax
from jax.experimental import pallas as pl
from jax.experimental.pallas import tpu as pltpu
from jax.experimental.pallas import tpu_sc as plsc

N = 10000
E = 320000
F = 128
H = 128
L = 64
D = 256
B = 16
NC = 6

NPAD = 10112
PAD_ROW = 10008
NCORES = 2
NSUB = 16
NTILES = NCORES * NSUB
CHUNK = 128
NCHUNK = 80
HCHUNK = NCHUNK // 2
HPAIR = HCHUNK // 2
EPT = NCHUNK * CHUNK
EPAD = NTILES * EPT
RPT = NPAD // NSUB

_MESH = plsc.VectorSubcoreMesh(core_axis_name="c", subcore_axis_name="s")


def _deg_kernel(dst_hbm, zeros_hbm, out_hbm, dst_v, ones_v, acc):
    cid = lax.axis_index("c")
    sid = lax.axis_index("s")
    wid = sid * NCORES + cid
    pltpu.sync_copy(dst_hbm.at[wid], dst_v)

    def init_ones(i, carry):
        ones_v[i, :] = jnp.ones((16,), jnp.float32)
        return carry

    lax.fori_loop(0, CHUNK, init_ones, 0)
    r0 = sid * RPT
    pltpu.sync_copy(zeros_hbm.at[pl.ds(r0, RPT)], acc.at[pl.ds(r0, RPT)])
    plsc.subcore_barrier()

    def body(j, carry):
        pltpu.sync_copy(ones_v, acc.at[dst_v.at[j]], add=True)
        return carry

    lax.fori_loop(0, NCHUNK, body, 0)
    plsc.subcore_barrier()
    pltpu.sync_copy(acc.at[pl.ds(r0, RPT)], out_hbm.at[cid, pl.ds(r0, RPT)])


_deg_call = pl.kernel(
    _deg_kernel,
    out_type=jax.ShapeDtypeStruct((NCORES, NPAD, 16), jnp.float32),
    mesh=_MESH,
    scratch_types=[
        pltpu.VMEM((NCHUNK, CHUNK), jnp.int32),
        pltpu.VMEM((CHUNK, 16), jnp.float32),
        pltpu.VMEM_SHARED((NPAD, 16), jnp.float32),
    ],
)


def _make_agg(width):
    def _agg_kernel(hp_hbm, src_hbm, dst_hbm, zeros_hbm, out_hbm,
                    src_v, dst_v, rows_v, acc, sema):
        cid = lax.axis_index("c")
        sid = lax.axis_index("s")
        wid = sid * NCORES + cid
        pltpu.sync_copy(src_hbm.at[wid], src_v)
        pltpu.sync_copy(dst_hbm.at[wid], dst_v)
        r0 = sid * RPT
        pltpu.sync_copy(zeros_hbm.at[pl.ds(r0, RPT)], acc.at[pl.ds(r0, RPT)])
        plsc.subcore_barrier()

        def body(j, carry):
            pltpu.async_copy(hp_hbm.at[src_v.at[j]], rows_v, sema).wait()
            pltpu.sync_copy(rows_v, acc.at[dst_v.at[j]], add=True)
            return carry

        lax.fori_loop(0, NCHUNK, body, 0)
        plsc.subcore_barrier()
        pltpu.sync_copy(acc.at[pl.ds(r0, RPT)], out_hbm.at[cid, pl.ds(r0, RPT)])

    return pl.kernel(
        _agg_kernel,
        out_type=jax.ShapeDtypeStruct((NCORES, NPAD, width), jnp.float32),
        mesh=_MESH,
        scratch_types=[
            pltpu.VMEM((NCHUNK, CHUNK), jnp.int32),
            pltpu.VMEM((NCHUNK, CHUNK), jnp.int32),
            pltpu.VMEM((CHUNK, width), jnp.float32),
            pltpu.VMEM_SHARED((NPAD, width), jnp.float32),
            pltpu.SemaphoreType.DMA,
        ],
    )


_agg_h = _make_agg(H)


def _tc0_body(degp_ref, x_ref, w1_ref, dinv_ref, h1p_ref):
    deg = degp_ref[0][:, 0:1] + degp_ref[1][:, 0:1] + 1.0
    dinv = lax.rsqrt(deg)
    dinv_ref[...] = jnp.broadcast_to(dinv, (NPAD, 16))
    h1p_ref[...] = dinv * jnp.dot(x_ref[...], w1_ref[...],
                                  preferred_element_type=jnp.float32)


def _tc0(degp, x_p, w1):
    return pl.pallas_call(
        _tc0_body,
        out_shape=(
            jax.ShapeDtypeStruct((NPAD, 16), jnp.float32),
            jax.ShapeDtypeStruct((NPAD, F), jnp.float32),
        ),
    )(degp, x_p, w1)


def _tc_mid_body(sp_ref, hp_ref, dinv_ref, b_ref, g_ref, be_ref, w_ref, out_ref):
    dinv = dinv_ref[...][:, 0:1]
    pre = dinv * (sp_ref[0] + sp_ref[1] + hp_ref[...]) + b_ref[...]
    m = jnp.mean(pre[:N], axis=0, keepdims=True)
    c = pre - m
    v = jnp.mean(c[:N] * c[:N], axis=0, keepdims=True)
    a = jnp.maximum(c * lax.rsqrt(v + 1e-5) * g_ref[...] + be_ref[...], 0.0)
    mask = (lax.broadcasted_iota(jnp.int32, (NPAD, 1), 0) < N).astype(jnp.float32)
    out_ref[...] = dinv * jnp.dot(a * mask, w_ref[...],
                                  preferred_element_type=jnp.float32)


def _tc_mid(sp, hp, dinv16, b, g, be, w, width_out):
    return pl.pallas_call(
        _tc_mid_body,
        out_shape=jax.ShapeDtypeStruct((NPAD, width_out), jnp.float32),
    )(sp, hp, dinv16, b, g, be, w)


def _tc_fin_body(sp_ref, hp_ref, dinv_ref, b3_ref, g3_ref, be3_ref,
                 batch_ref, cam_ref, wl0a_ref, wl0b_ref, bl0_ref,
                 wl1_ref, bl1_ref, wl2_ref, bl2_ref, wout_ref, bout_ref,
                 out_ref):
    dinv = dinv_ref[...][:, 0:1]
    pre = dinv * (sp_ref[0] + sp_ref[1] + hp_ref[...]) + b3_ref[...]
    m = jnp.mean(pre[:N], axis=0, keepdims=True)
    c = pre - m
    v = jnp.mean(c[:N] * c[:N], axis=0, keepdims=True)
    a = c * lax.rsqrt(v + 1e-5) * g3_ref[...] + be3_ref[...]
    rid = lax.broadcasted_iota(jnp.int32, (B, NPAD), 0)
    p = (rid == batch_ref[...]).astype(jnp.float32)
    sums = jnp.dot(p, a, preferred_element_type=jnp.float32)
    cnt = jnp.sum(p, axis=1, keepdims=True)
    pooled = sums / jnp.maximum(cnt, 1.0)
    xd = jnp.maximum(
        jnp.dot(pooled, wl0a_ref[...], preferred_element_type=jnp.float32)
        + jnp.dot(cam_ref[...], wl0b_ref[...], preferred_element_type=jnp.float32)
        + bl0_ref[...], 0.0)
    xd = jnp.maximum(
        jnp.dot(xd, wl1_ref[...], preferred_element_type=jnp.float32)
        + bl1_ref[...], 0.0)
    xd = jnp.maximum(
        jnp.dot(xd, wl2_ref[...], preferred_element_type=jnp.float32)
        + bl2_ref[...], 0.0)
    out_ref[...] = (jnp.dot(xd, wout_ref[...], preferred_element_type=jnp.float32)
                    + bout_ref[...])


def _tc_fin(sp, hp, dinv16, b3, g3, be3, batch_p, cam,
            wl0a, wl0b, bl0, wl1, bl1, wl2, bl2, wout, bout):
    return pl.pallas_call(
        _tc_fin_body,
        out_shape=jax.ShapeDtypeStruct((B, 3), jnp.float32),
    )(sp, hp, dinv16, b3, g3, be3, batch_p, cam,
      wl0a, wl0b, bl0, wl1, bl1, wl2, bl2, wout, bout)


def kernel(x, edge_index, origin, direction, batch,
           W1, b1, g1, be1, W2, b2, g2, be2, W3, b3, g3, be3,
           Wl0, bl0, Wl1, bl1, Wl2, bl2, Wout, bout):
    src = edge_index[0].astype(jnp.int32)
    dst = edge_index[1].astype(jnp.int32)
    pad = jnp.full((EPAD - E,), PAD_ROW, jnp.int32)
    src3 = jnp.concatenate([src, pad]).reshape(NTILES, NCHUNK, CHUNK)
    dst3 = jnp.concatenate([dst, pad]).reshape(NTILES, NCHUNK, CHUNK)
    x_p = jnp.pad(x, ((0, NPAD - N), (0, 0)))
    batch_p = jnp.pad(batch.astype(jnp.int32), (0, NPAD - N),
                      constant_values=-1).reshape(1, NPAD)
    cam = jnp.concatenate([origin, direction], axis=1)
    z16 = jnp.zeros((NPAD, 16), jnp.float32)
    zh = jnp.zeros((NPAD, H), jnp.float32)
    W3p = jnp.pad(W3, ((0, 0), (0, H - L)))
    b3p = jnp.pad(b3, (0, H - L)).reshape(1, H)
    g3p = jnp.pad(g3, (0, H - L)).reshape(1, H)
    be3p = jnp.pad(be3, (0, H - L)).reshape(1, H)
    wl0a = jnp.pad(Wl0[:L], ((0, H - L), (0, 0)))

    degp = _deg_call(dst3, z16)
    dinv16, h1p = _tc0(degp, x_p, W1)
    s1 = _agg_h(h1p, src3, dst3, zh)
    h2p = _tc_mid(s1, h1p, dinv16, b1.reshape(1, H), g1.reshape(1, H),
                  be1.reshape(1, H), W2, H)
    s2 = _agg_h(h2p, src3, dst3, zh)
    h3p = _tc_mid(s2, h2p, dinv16, b2.reshape(1, H), g2.reshape(1, H),
                  be2.reshape(1, H), W3p, H)
    s3 = _agg_h(h3p, src3, dst3, zh)
    return _tc_fin(s3, h3p, dinv16, b3p, g3p, be3p, batch_p, cam,
                   wl0a, Wl0[L:], bl0.reshape(1, D),
                   Wl1, bl1.reshape(1, D), Wl2, bl2.reshape(1, D),
                   Wout, bout.reshape(1, 3))


# speedup vs baseline: 2.7511x; 2.6451x over previous
import functools

import jax
import jax.numpy as jnp
from jax import lax
from jax.experimental import pallas as pl
from jax.experimental.pallas import tpu as pltpu
from jax.experimental.pallas import tpu_sc as plsc

N = 10000
E = 320000
F = 128
H = 128
L = 64
D = 256
B = 16
NC = 6

NPAD = 10112
PAD_ROW = 10008
NCORES = 2
NSUB = 16
NTILES = NCORES * NSUB
CHUNK = 128
NCHUNK = 80
HCHUNK = NCHUNK // 2
HPAIR = HCHUNK // 2
EPT = NCHUNK * CHUNK
EPAD = NTILES * EPT
RPT = NPAD // NSUB

_MESH = plsc.VectorSubcoreMesh(core_axis_name="c", subcore_axis_name="s")


def _deg_kernel(dst_hbm, zeros_hbm, out_hbm, dst_v, ones_v, acc):
    cid = lax.axis_index("c")
    sid = lax.axis_index("s")
    wid = sid * NCORES + cid
    pltpu.sync_copy(dst_hbm.at[wid], dst_v)

    def init_ones(i, carry):
        ones_v[i, :] = jnp.ones((16,), jnp.float32)
        return carry

    lax.fori_loop(0, CHUNK, init_ones, 0)
    r0 = sid * RPT
    pltpu.sync_copy(zeros_hbm.at[pl.ds(r0, RPT)], acc.at[pl.ds(r0, RPT)])
    plsc.subcore_barrier()

    def body(j, carry):
        pltpu.sync_copy(ones_v, acc.at[dst_v.at[j]], add=True)
        return carry

    lax.fori_loop(0, NCHUNK, body, 0)
    plsc.subcore_barrier()
    pltpu.sync_copy(acc.at[pl.ds(r0, RPT)], out_hbm.at[cid, pl.ds(r0, RPT)])


_deg_call = pl.kernel(
    _deg_kernel,
    out_type=jax.ShapeDtypeStruct((NCORES, NPAD, 16), jnp.float32),
    mesh=_MESH,
    scratch_types=[
        pltpu.VMEM((NCHUNK, CHUNK), jnp.int32),
        pltpu.VMEM((CHUNK, 16), jnp.float32),
        pltpu.VMEM_SHARED((NPAD, 16), jnp.float32),
    ],
)


def _make_agg(width):
    def _agg_kernel(hp_hbm, src_hbm, dst_hbm, zeros_hbm, out_hbm,
                    src_v, dst_v, rows_v, acc, sema):
        cid = lax.axis_index("c")
        sid = lax.axis_index("s")
        wid = sid * NCORES + cid
        pltpu.sync_copy(src_hbm.at[wid], src_v)
        pltpu.sync_copy(dst_hbm.at[wid], dst_v)
        r0 = sid * RPT
        pltpu.sync_copy(zeros_hbm.at[pl.ds(r0, RPT)], acc.at[pl.ds(r0, RPT)])
        plsc.subcore_barrier()

        def body(j, carry):
            pltpu.async_copy(hp_hbm.at[src_v.at[j]], rows_v, sema).wait()
            pltpu.sync_copy(rows_v, acc.at[dst_v.at[j]], add=True)
            return carry

        lax.fori_loop(0, NCHUNK, body, 0)
        plsc.subcore_barrier()
        pltpu.sync_copy(acc.at[pl.ds(r0, RPT)], out_hbm.at[cid, pl.ds(r0, RPT)])

    return pl.kernel(
        _agg_kernel,
        out_type=jax.ShapeDtypeStruct((NCORES, NPAD, width), jnp.float32),
        mesh=_MESH,
        scratch_types=[
            pltpu.VMEM((NCHUNK, CHUNK), jnp.int32),
            pltpu.VMEM((NCHUNK, CHUNK), jnp.int32),
            pltpu.VMEM((CHUNK, width), jnp.float32),
            pltpu.VMEM_SHARED((NPAD, width), jnp.float32),
            pltpu.SemaphoreType.DMA,
        ],
    )


_agg_h = _make_agg(H)


def _tc0_body(degp_ref, x_ref, w1_ref, dinv_ref, h1p_ref):
    deg = degp_ref[0][:, 0:1] + degp_ref[1][:, 0:1] + 1.0
    dinv = lax.rsqrt(deg)
    dinv_ref[...] = jnp.broadcast_to(dinv, (NPAD, 16))
    h1p_ref[...] = dinv * jnp.dot(x_ref[...], w1_ref[...],
                                  preferred_element_type=jnp.float32)


def _tc0(degp, x_p, w1):
    return pl.pallas_call(
        _tc0_body,
        out_shape=(
            jax.ShapeDtypeStruct((NPAD, 16), jnp.float32),
            jax.ShapeDtypeStruct((NPAD, F), jnp.float32),
        ),
    )(degp, x_p, w1)


def _tc_mid_body(sp_ref, hp_ref, dinv_ref, b_ref, g_ref, be_ref, w_ref, out_ref):
    dinv = dinv_ref[...][:, 0:1]
    pre = dinv * (sp_ref[0] + sp_ref[1] + hp_ref[...]) + b_ref[...]
    m = jnp.mean(pre[:N], axis=0, keepdims=True)
    c = pre - m
    v = jnp.mean(c[:N] * c[:N], axis=0, keepdims=True)
    a = jnp.maximum(c * lax.rsqrt(v + 1e-5) * g_ref[...] + be_ref[...], 0.0)
    mask = (lax.broadcasted_iota(jnp.int32, (NPAD, 1), 0) < N).astype(jnp.float32)
    out_ref[...] = dinv * jnp.dot(a * mask, w_ref[...],
                                  preferred_element_type=jnp.float32)


def _tc_mid(sp, hp, dinv16, b, g, be, w, width_out):
    return pl.pallas_call(
        _tc_mid_body,
        out_shape=jax.ShapeDtypeStruct((NPAD, width_out), jnp.float32),
    )(sp, hp, dinv16, b, g, be, w)


def _tc_fin_body(sp_ref, hp_ref, dinv_ref, b3_ref, g3_ref, be3_ref,
                 batch_ref, cam_ref, wl0a_ref, wl0b_ref, bl0_ref,
                 wl1_ref, bl1_ref, wl2_ref, bl2_ref, wout_ref, bout_ref,
                 out_ref):
    dinv = dinv_ref[...][:, 0:1]
    pre = dinv * (sp_ref[0] + sp_ref[1] + hp_ref[...]) + b3_ref[...]
    m = jnp.mean(pre[:N], axis=0, keepdims=True)
    c = pre - m
    v = jnp.mean(c[:N] * c[:N], axis=0, keepdims=True)
    a = c * lax.rsqrt(v + 1e-5) * g3_ref[...] + be3_ref[...]
    rid = lax.broadcasted_iota(jnp.int32, (B, NPAD), 0)
    p = (rid == batch_ref[...]).astype(jnp.float32)
    sums = jnp.dot(p, a, preferred_element_type=jnp.float32)
    cnt = jnp.sum(p, axis=1, keepdims=True)
    pooled = sums / jnp.maximum(cnt, 1.0)
    xd = jnp.maximum(
        jnp.dot(pooled, wl0a_ref[...], preferred_element_type=jnp.float32)
        + jnp.dot(cam_ref[...], wl0b_ref[...], preferred_element_type=jnp.float32)
        + bl0_ref[...], 0.0)
    xd = jnp.maximum(
        jnp.dot(xd, wl1_ref[...], preferred_element_type=jnp.float32)
        + bl1_ref[...], 0.0)
    xd = jnp.maximum(
        jnp.dot(xd, wl2_ref[...], preferred_element_type=jnp.float32)
        + bl2_ref[...], 0.0)
    out_ref[...] = (jnp.dot(xd, wout_ref[...], preferred_element_type=jnp.float32)
                    + bout_ref[...])


def _tc_fin(sp, hp, dinv16, b3, g3, be3, batch_p, cam,
            wl0a, wl0b, bl0, wl1, bl1, wl2, bl2, wout, bout):
    return pl.pallas_call(
        _tc_fin_body,
        out_shape=jax.ShapeDtypeStruct((B, 3), jnp.float32),
    )(sp, hp, dinv16, b3, g3, be3, batch_p, cam,
      wl0a, wl0b, bl0, wl1, bl1, wl2, bl2, wout, bout)


def kernel(x, edge_index, origin, direction, batch,
           W1, b1, g1, be1, W2, b2, g2, be2, W3, b3, g3, be3,
           Wl0, bl0, Wl1, bl1, Wl2, bl2, Wout, bout):
    src = edge_index[0].astype(jnp.int32)
    dst = edge_index[1].astype(jnp.int32)
    pad = N + jnp.arange(EPAD - E, dtype=jnp.int32) % (NPAD - N)
    src3 = jnp.concatenate([src, pad]).reshape(NTILES, NCHUNK, CHUNK)
    dst3 = jnp.concatenate([dst, pad]).reshape(NTILES, NCHUNK, CHUNK)
    x_p = jnp.pad(x, ((0, NPAD - N), (0, 0)))
    batch_p = jnp.pad(batch.astype(jnp.int32), (0, NPAD - N),
                      constant_values=-1).reshape(1, NPAD)
    cam = jnp.concatenate([origin, direction], axis=1)
    z16 = jnp.zeros((NPAD, 16), jnp.float32)
    zh = jnp.zeros((NPAD, H), jnp.float32)
    W3p = jnp.pad(W3, ((0, 0), (0, H - L)))
    b3p = jnp.pad(b3, (0, H - L)).reshape(1, H)
    g3p = jnp.pad(g3, (0, H - L)).reshape(1, H)
    be3p = jnp.pad(be3, (0, H - L)).reshape(1, H)
    wl0a = jnp.pad(Wl0[:L], ((0, H - L), (0, 0)))

    degp = _deg_call(dst3, z16)
    dinv16, h1p = _tc0(degp, x_p, W1)
    s1 = _agg_h(h1p, src3, dst3, zh)
    h2p = _tc_mid(s1, h1p, dinv16, b1.reshape(1, H), g1.reshape(1, H),
                  be1.reshape(1, H), W2, H)
    s2 = _agg_h(h2p, src3, dst3, zh)
    h3p = _tc_mid(s2, h2p, dinv16, b2.reshape(1, H), g2.reshape(1, H),
                  be2.reshape(1, H), W3p, H)
    s3 = _agg_h(h3p, src3, dst3, zh)
    return _tc_fin(s3, h3p, dinv16, b3p, g3p, be3p, batch_p, cam,
                   wl0a, Wl0[L:], bl0.reshape(1, D),
                   Wl1, bl1.reshape(1, D), Wl2, bl2.reshape(1, D),
                   Wout, bout.reshape(1, 3))


# speedup vs baseline: 3.0550x; 1.1105x over previous
import functools

import jax
import jax.numpy as jnp
from jax import lax
from jax.experimental import pallas as pl
from jax.experimental.pallas import tpu as pltpu
from jax.experimental.pallas import tpu_sc as plsc

N = 10000
E = 320000
F = 128
H = 128
L = 64
D = 256
B = 16
NC = 6

NPAD = 10112
PAD_ROW = 10008
NCORES = 2
NSUB = 16
NTILES = NCORES * NSUB
CHUNK = 128
NCHUNK = 80
HCHUNK = NCHUNK // 2
HPAIR = HCHUNK // 2
EPT = NCHUNK * CHUNK
EPAD = NTILES * EPT
RPT = NPAD // NSUB

_MESH = plsc.VectorSubcoreMesh(core_axis_name="c", subcore_axis_name="s")


def _deg_kernel(dst_hbm, zeros_hbm, out_hbm, dst_v, ones_v, acc):
    cid = lax.axis_index("c")
    sid = lax.axis_index("s")
    wid = sid * NCORES + cid
    pltpu.sync_copy(dst_hbm.at[wid], dst_v)

    def init_ones(i, carry):
        ones_v[i, :] = jnp.ones((16,), jnp.float32)
        return carry

    lax.fori_loop(0, CHUNK, init_ones, 0)
    r0 = sid * RPT
    pltpu.sync_copy(zeros_hbm.at[pl.ds(r0, RPT)], acc.at[pl.ds(r0, RPT)])
    plsc.subcore_barrier()

    def body(j, carry):
        pltpu.sync_copy(ones_v, acc.at[dst_v.at[j]], add=True)
        return carry

    lax.fori_loop(0, NCHUNK, body, 0)
    plsc.subcore_barrier()
    pltpu.sync_copy(acc.at[pl.ds(r0, RPT)], out_hbm.at[cid, pl.ds(r0, RPT)])


_deg_call = pl.kernel(
    _deg_kernel,
    out_type=jax.ShapeDtypeStruct((NCORES, NPAD, 16), jnp.float32),
    mesh=_MESH,
    scratch_types=[
        pltpu.VMEM((NCHUNK, CHUNK), jnp.int32),
        pltpu.VMEM((CHUNK, 16), jnp.float32),
        pltpu.VMEM_SHARED((NPAD, 16), jnp.float32),
    ],
)


def _make_agg(width):
    def _agg_kernel(hp_hbm, src_hbm, dst_hbm, zeros_hbm, out_hbm,
                    src_v, dst_v, rows_v, rows_b, acc, sema):
        cid = lax.axis_index("c")
        sid = lax.axis_index("s")
        wid = sid * NCORES + cid
        r0 = sid * RPT
        pltpu.sync_copy(zeros_hbm.at[pl.ds(r0, RPT)], acc.at[pl.ds(r0, RPT)])
        plsc.subcore_barrier()

        for half in range(2):
            pltpu.sync_copy(src_hbm.at[wid, pl.ds(half * HCHUNK, HCHUNK)],
                            src_v)
            pltpu.sync_copy(dst_hbm.at[wid, pl.ds(half * HCHUNK, HCHUNK)],
                            dst_v)

            def pair(i, carry):
                c0 = 2 * i
                da = pltpu.async_copy(hp_hbm.at[src_v.at[c0]], rows_v, sema)
                db = pltpu.async_copy(hp_hbm.at[src_v.at[c0 + 1]], rows_b,
                                      sema)
                da.wait()
                db.wait()
                pltpu.sync_copy(rows_v, acc.at[dst_v.at[c0]], add=True)
                pltpu.sync_copy(rows_b, acc.at[dst_v.at[c0 + 1]], add=True)
                return carry

            lax.fori_loop(0, HPAIR, pair, 0)
        plsc.subcore_barrier()
        pltpu.sync_copy(acc.at[pl.ds(r0, RPT)], out_hbm.at[cid, pl.ds(r0, RPT)])

    return pl.kernel(
        _agg_kernel,
        out_type=jax.ShapeDtypeStruct((NCORES, NPAD, width), jnp.float32),
        mesh=_MESH,
        scratch_types=[
            pltpu.VMEM((HCHUNK, CHUNK), jnp.int32),
            pltpu.VMEM((HCHUNK, CHUNK), jnp.int32),
            pltpu.VMEM((CHUNK, width), jnp.float32),
            pltpu.VMEM((CHUNK, width), jnp.float32),
            pltpu.VMEM_SHARED((NPAD, width), jnp.float32),
            pltpu.SemaphoreType.DMA,
        ],
    )


_agg_h = _make_agg(H)


def _tc0_body(degp_ref, x_ref, w1_ref, dinv_ref, h1p_ref):
    deg = degp_ref[0][:, 0:1] + degp_ref[1][:, 0:1] + 1.0
    dinv = lax.rsqrt(deg)
    dinv_ref[...] = jnp.broadcast_to(dinv, (NPAD, 16))
    h1p_ref[...] = dinv * jnp.dot(x_ref[...], w1_ref[...],
                                  preferred_element_type=jnp.float32)


def _tc0(degp, x_p, w1):
    return pl.pallas_call(
        _tc0_body,
        out_shape=(
            jax.ShapeDtypeStruct((NPAD, 16), jnp.float32),
            jax.ShapeDtypeStruct((NPAD, F), jnp.float32),
        ),
    )(degp, x_p, w1)


def _tc_mid_body(sp_ref, hp_ref, dinv_ref, b_ref, g_ref, be_ref, w_ref, out_ref):
    dinv = dinv_ref[...][:, 0:1]
    pre = dinv * (sp_ref[0] + sp_ref[1] + hp_ref[...]) + b_ref[...]
    m = jnp.mean(pre[:N], axis=0, keepdims=True)
    c = pre - m
    v = jnp.mean(c[:N] * c[:N], axis=0, keepdims=True)
    a = jnp.maximum(c * lax.rsqrt(v + 1e-5) * g_ref[...] + be_ref[...], 0.0)
    mask = (lax.broadcasted_iota(jnp.int32, (NPAD, 1), 0) < N).astype(jnp.float32)
    out_ref[...] = dinv * jnp.dot(a * mask, w_ref[...],
                                  preferred_element_type=jnp.float32)


def _tc_mid(sp, hp, dinv16, b, g, be, w, width_out):
    return pl.pallas_call(
        _tc_mid_body,
        out_shape=jax.ShapeDtypeStruct((NPAD, width_out), jnp.float32),
    )(sp, hp, dinv16, b, g, be, w)


def _tc_fin_body(sp_ref, hp_ref, dinv_ref, b3_ref, g3_ref, be3_ref,
                 batch_ref, cam_ref, wl0a_ref, wl0b_ref, bl0_ref,
                 wl1_ref, bl1_ref, wl2_ref, bl2_ref, wout_ref, bout_ref,
                 out_ref):
    dinv = dinv_ref[...][:, 0:1]
    pre = dinv * (sp_ref[0] + sp_ref[1] + hp_ref[...]) + b3_ref[...]
    m = jnp.mean(pre[:N], axis=0, keepdims=True)
    c = pre - m
    v = jnp.mean(c[:N] * c[:N], axis=0, keepdims=True)
    a = c * lax.rsqrt(v + 1e-5) * g3_ref[...] + be3_ref[...]
    rid = lax.broadcasted_iota(jnp.int32, (B, NPAD), 0)
    p = (rid == batch_ref[...]).astype(jnp.float32)
    sums = jnp.dot(p, a, preferred_element_type=jnp.float32)
    cnt = jnp.sum(p, axis=1, keepdims=True)
    pooled = sums / jnp.maximum(cnt, 1.0)
    xd = jnp.maximum(
        jnp.dot(pooled, wl0a_ref[...], preferred_element_type=jnp.float32)
        + jnp.dot(cam_ref[...], wl0b_ref[...], preferred_element_type=jnp.float32)
        + bl0_ref[...], 0.0)
    xd = jnp.maximum(
        jnp.dot(xd, wl1_ref[...], preferred_element_type=jnp.float32)
        + bl1_ref[...], 0.0)
    xd = jnp.maximum(
        jnp.dot(xd, wl2_ref[...], preferred_element_type=jnp.float32)
        + bl2_ref[...], 0.0)
    out_ref[...] = (jnp.dot(xd, wout_ref[...], preferred_element_type=jnp.float32)
                    + bout_ref[...])


def _tc_fin(sp, hp, dinv16, b3, g3, be3, batch_p, cam,
            wl0a, wl0b, bl0, wl1, bl1, wl2, bl2, wout, bout):
    return pl.pallas_call(
        _tc_fin_body,
        out_shape=jax.ShapeDtypeStruct((B, 3), jnp.float32),
    )(sp, hp, dinv16, b3, g3, be3, batch_p, cam,
      wl0a, wl0b, bl0, wl1, bl1, wl2, bl2, wout, bout)


def kernel(x, edge_index, origin, direction, batch,
           W1, b1, g1, be1, W2, b2, g2, be2, W3, b3, g3, be3,
           Wl0, bl0, Wl1, bl1, Wl2, bl2, Wout, bout):
    src = edge_index[0].astype(jnp.int32)
    dst = edge_index[1].astype(jnp.int32)
    pad = N + jnp.arange(EPAD - E, dtype=jnp.int32) % (NPAD - N)
    src3 = jnp.concatenate([src, pad]).reshape(NTILES, NCHUNK, CHUNK)
    dst3 = jnp.concatenate([dst, pad]).reshape(NTILES, NCHUNK, CHUNK)
    x_p = jnp.pad(x, ((0, NPAD - N), (0, 0)))
    batch_p = jnp.pad(batch.astype(jnp.int32), (0, NPAD - N),
                      constant_values=-1).reshape(1, NPAD)
    cam = jnp.concatenate([origin, direction], axis=1)
    z16 = jnp.zeros((NPAD, 16), jnp.float32)
    zh = jnp.zeros((NPAD, H), jnp.float32)
    W3p = jnp.pad(W3, ((0, 0), (0, H - L)))
    b3p = jnp.pad(b3, (0, H - L)).reshape(1, H)
    g3p = jnp.pad(g3, (0, H - L)).reshape(1, H)
    be3p = jnp.pad(be3, (0, H - L)).reshape(1, H)
    wl0a = jnp.pad(Wl0[:L], ((0, H - L), (0, 0)))

    degp = _deg_call(dst3, z16)
    dinv16, h1p = _tc0(degp, x_p, W1)
    s1 = _agg_h(h1p, src3, dst3, zh)
    h2p = _tc_mid(s1, h1p, dinv16, b1.reshape(1, H), g1.reshape(1, H),
                  be1.reshape(1, H), W2, H)
    s2 = _agg_h(h2p, src3, dst3, zh)
    h3p = _tc_mid(s2, h2p, dinv16, b2.reshape(1, H), g2.reshape(1, H),
                  be2.reshape(1, H), W3p, H)
    s3 = _agg_h(h3p, src3, dst3, zh)
    return _tc_fin(s3, h3p, dinv16, b3p, g3p, be3p, batch_p, cam,
                   wl0a, Wl0[L:], bl0.reshape(1, D),
                   Wl1, bl1.reshape(1, D), Wl2, bl2.reshape(1, D),
                   Wout, bout.reshape(1, 3))


# speedup vs baseline: 3.4557x; 1.1311x over previous
import functools

import jax
import jax.numpy as jnp
from jax import lax
from jax.experimental import pallas as pl
from jax.experimental.pallas import tpu as pltpu
from jax.experimental.pallas import tpu_sc as plsc

N = 10000
E = 320000
F = 128
H = 128
L = 64
D = 256
B = 16
NC = 6

NPAD = 10112
PAD_ROW = 10008
NCORES = 2
NSUB = 16
NTILES = NCORES * NSUB
CHUNK = 128
NCHUNK = 80
HCHUNK = NCHUNK // 2
HPAIR = HCHUNK // 2
EPT = NCHUNK * CHUNK
EPAD = NTILES * EPT
RPT = NPAD // NSUB

_MESH = plsc.VectorSubcoreMesh(core_axis_name="c", subcore_axis_name="s")


def _deg_kernel(dst_hbm, zeros_hbm, out_hbm, dst_v, ones_v, acc):
    cid = lax.axis_index("c")
    sid = lax.axis_index("s")
    wid = sid * NCORES + cid
    pltpu.sync_copy(dst_hbm.at[wid], dst_v)

    def init_ones(i, carry):
        ones_v[i, :] = jnp.ones((16,), jnp.float32)
        return carry

    lax.fori_loop(0, CHUNK, init_ones, 0)
    r0 = sid * RPT
    pltpu.sync_copy(zeros_hbm.at[pl.ds(r0, RPT)], acc.at[pl.ds(r0, RPT)])
    plsc.subcore_barrier()

    def body(j, carry):
        pltpu.sync_copy(ones_v, acc.at[dst_v.at[j]], add=True)
        return carry

    lax.fori_loop(0, NCHUNK, body, 0)
    plsc.subcore_barrier()
    pltpu.sync_copy(acc.at[pl.ds(r0, RPT)], out_hbm.at[cid, pl.ds(r0, RPT)])


_deg_call = pl.kernel(
    _deg_kernel,
    out_type=jax.ShapeDtypeStruct((NCORES, NPAD, 16), jnp.float32),
    mesh=_MESH,
    scratch_types=[
        pltpu.VMEM((NCHUNK, CHUNK), jnp.int32),
        pltpu.VMEM((CHUNK, 16), jnp.float32),
        pltpu.VMEM_SHARED((NPAD, 16), jnp.float32),
    ],
)


def _make_agg(width):
    def _agg_kernel(hp_hbm, src_hbm, dst_hbm, zeros_hbm, out_hbm,
                    src_v, dst_v, rows_v, rows_b, acc, sema):
        cid = lax.axis_index("c")
        sid = lax.axis_index("s")
        wid = sid * NCORES + cid
        r0 = sid * RPT
        pltpu.sync_copy(zeros_hbm.at[pl.ds(r0, RPT)], acc.at[pl.ds(r0, RPT)])
        plsc.subcore_barrier()

        for half in range(2):
            pltpu.sync_copy(src_hbm.at[wid, pl.ds(half * HCHUNK, HCHUNK)],
                            src_v)
            pltpu.sync_copy(dst_hbm.at[wid, pl.ds(half * HCHUNK, HCHUNK)],
                            dst_v)

            def pair(i, carry):
                c0 = 2 * i
                da = pltpu.async_copy(hp_hbm.at[src_v.at[c0]], rows_v, sema)
                db = pltpu.async_copy(hp_hbm.at[src_v.at[c0 + 1]], rows_b,
                                      sema)
                da.wait()
                pltpu.sync_copy(rows_v, acc.at[dst_v.at[c0]], add=True)
                db.wait()
                pltpu.sync_copy(rows_b, acc.at[dst_v.at[c0 + 1]], add=True)
                return carry

            lax.fori_loop(0, HPAIR, pair, 0)
        plsc.subcore_barrier()
        pltpu.sync_copy(acc.at[pl.ds(r0, RPT)], out_hbm.at[cid, pl.ds(r0, RPT)])

    return pl.kernel(
        _agg_kernel,
        out_type=jax.ShapeDtypeStruct((NCORES, NPAD, width), jnp.float32),
        mesh=_MESH,
        scratch_types=[
            pltpu.VMEM((HCHUNK, CHUNK), jnp.int32),
            pltpu.VMEM((HCHUNK, CHUNK), jnp.int32),
            pltpu.VMEM((CHUNK, width), jnp.float32),
            pltpu.VMEM((CHUNK, width), jnp.float32),
            pltpu.VMEM_SHARED((NPAD, width), jnp.float32),
            pltpu.SemaphoreType.DMA,
        ],
    )


_agg_h = _make_agg(H)


def _tc0_body(degp_ref, x_ref, w1_ref, dinv_ref, h1p_ref):
    deg = degp_ref[0][:, 0:1] + degp_ref[1][:, 0:1] + 1.0
    dinv = lax.rsqrt(deg)
    dinv_ref[...] = jnp.broadcast_to(dinv, (NPAD, 16))
    h1p_ref[...] = dinv * jnp.dot(x_ref[...], w1_ref[...],
                                  preferred_element_type=jnp.float32)


def _tc0(degp, x_p, w1):
    return pl.pallas_call(
        _tc0_body,
        out_shape=(
            jax.ShapeDtypeStruct((NPAD, 16), jnp.float32),
            jax.ShapeDtypeStruct((NPAD, F), jnp.float32),
        ),
    )(degp, x_p, w1)


def _tc_mid_body(sp_ref, hp_ref, dinv_ref, b_ref, g_ref, be_ref, w_ref, out_ref):
    dinv = dinv_ref[...][:, 0:1]
    pre = dinv * (sp_ref[0] + sp_ref[1] + hp_ref[...]) + b_ref[...]
    m = jnp.mean(pre[:N], axis=0, keepdims=True)
    c = pre - m
    v = jnp.mean(c[:N] * c[:N], axis=0, keepdims=True)
    a = jnp.maximum(c * lax.rsqrt(v + 1e-5) * g_ref[...] + be_ref[...], 0.0)
    mask = (lax.broadcasted_iota(jnp.int32, (NPAD, 1), 0) < N).astype(jnp.float32)
    out_ref[...] = dinv * jnp.dot(a * mask, w_ref[...],
                                  preferred_element_type=jnp.float32)


def _tc_mid(sp, hp, dinv16, b, g, be, w, width_out):
    return pl.pallas_call(
        _tc_mid_body,
        out_shape=jax.ShapeDtypeStruct((NPAD, width_out), jnp.float32),
    )(sp, hp, dinv16, b, g, be, w)


def _tc_fin_body(sp_ref, hp_ref, dinv_ref, b3_ref, g3_ref, be3_ref,
                 batch_ref, cam_ref, wl0a_ref, wl0b_ref, bl0_ref,
                 wl1_ref, bl1_ref, wl2_ref, bl2_ref, wout_ref, bout_ref,
                 out_ref):
    dinv = dinv_ref[...][:, 0:1]
    pre = dinv * (sp_ref[0] + sp_ref[1] + hp_ref[...]) + b3_ref[...]
    m = jnp.mean(pre[:N], axis=0, keepdims=True)
    c = pre - m
    v = jnp.mean(c[:N] * c[:N], axis=0, keepdims=True)
    a = c * lax.rsqrt(v + 1e-5) * g3_ref[...] + be3_ref[...]
    rid = lax.broadcasted_iota(jnp.int32, (B, NPAD), 0)
    p = (rid == batch_ref[...]).astype(jnp.float32)
    sums = jnp.dot(p, a, preferred_element_type=jnp.float32)
    cnt = jnp.sum(p, axis=1, keepdims=True)
    pooled = sums / jnp.maximum(cnt, 1.0)
    xd = jnp.maximum(
        jnp.dot(pooled, wl0a_ref[...], preferred_element_type=jnp.float32)
        + jnp.dot(cam_ref[...], wl0b_ref[...], preferred_element_type=jnp.float32)
        + bl0_ref[...], 0.0)
    xd = jnp.maximum(
        jnp.dot(xd, wl1_ref[...], preferred_element_type=jnp.float32)
        + bl1_ref[...], 0.0)
    xd = jnp.maximum(
        jnp.dot(xd, wl2_ref[...], preferred_element_type=jnp.float32)
        + bl2_ref[...], 0.0)
    out_ref[...] = (jnp.dot(xd, wout_ref[...], preferred_element_type=jnp.float32)
                    + bout_ref[...])


def _tc_fin(sp, hp, dinv16, b3, g3, be3, batch_p, cam,
            wl0a, wl0b, bl0, wl1, bl1, wl2, bl2, wout, bout):
    return pl.pallas_call(
        _tc_fin_body,
        out_shape=jax.ShapeDtypeStruct((B, 3), jnp.float32),
    )(sp, hp, dinv16, b3, g3, be3, batch_p, cam,
      wl0a, wl0b, bl0, wl1, bl1, wl2, bl2, wout, bout)


def kernel(x, edge_index, origin, direction, batch,
           W1, b1, g1, be1, W2, b2, g2, be2, W3, b3, g3, be3,
           Wl0, bl0, Wl1, bl1, Wl2, bl2, Wout, bout):
    src = edge_index[0].astype(jnp.int32)
    dst = edge_index[1].astype(jnp.int32)
    pad = N + jnp.arange(EPAD - E, dtype=jnp.int32) % (NPAD - N)
    src3 = jnp.concatenate([src, pad]).reshape(NTILES, NCHUNK, CHUNK)
    dst3 = jnp.concatenate([dst, pad]).reshape(NTILES, NCHUNK, CHUNK)
    x_p = jnp.pad(x, ((0, NPAD - N), (0, 0)))
    batch_p = jnp.pad(batch.astype(jnp.int32), (0, NPAD - N),
                      constant_values=-1).reshape(1, NPAD)
    cam = jnp.concatenate([origin, direction], axis=1)
    z16 = jnp.zeros((NPAD, 16), jnp.float32)
    zh = jnp.zeros((NPAD, H), jnp.float32)
    W3p = jnp.pad(W3, ((0, 0), (0, H - L)))
    b3p = jnp.pad(b3, (0, H - L)).reshape(1, H)
    g3p = jnp.pad(g3, (0, H - L)).reshape(1, H)
    be3p = jnp.pad(be3, (0, H - L)).reshape(1, H)
    wl0a = jnp.pad(Wl0[:L], ((0, H - L), (0, 0)))

    degp = _deg_call(dst3, z16)
    dinv16, h1p = _tc0(degp, x_p, W1)
    s1 = _agg_h(h1p, src3, dst3, zh)
    h2p = _tc_mid(s1, h1p, dinv16, b1.reshape(1, H), g1.reshape(1, H),
                  be1.reshape(1, H), W2, H)
    s2 = _agg_h(h2p, src3, dst3, zh)
    h3p = _tc_mid(s2, h2p, dinv16, b2.reshape(1, H), g2.reshape(1, H),
                  be2.reshape(1, H), W3p, H)
    s3 = _agg_h(h3p, src3, dst3, zh)
    return _tc_fin(s3, h3p, dinv16, b3p, g3p, be3p, batch_p, cam,
                   wl0a, Wl0[L:], bl0.reshape(1, D),
                   Wl1, bl1.reshape(1, D), Wl2, bl2.reshape(1, D),
                   Wout, bout.reshape(1, 3))
